# baseline (speedup 1.0000x reference)
import functools
import jax
import jax.numpy as jnp
from jax import lax
from jax.experimental import pallas as pl
from jax.experimental.pallas import tpu as pltpu
from jax.experimental.pallas import tpu_sc as plsc

MAX_SIZE = 1000
ZCHUNK = 64000
DRAIN = 13


def kernel(x):
    B, F = x.shape
    nc, ns = 2, 16
    nw = nc * ns
    n = B * F
    total = n * MAX_SIZE
    words_pw = total // nw
    nz = words_pw // ZCHUNK
    f_per_c = F // nc
    b_per_s = B // ns
    pairs_pw = f_per_c * b_per_s
    irows = pairs_pw // 128

    bi = jnp.arange(B, dtype=jnp.int32)[:, None]
    fi = jnp.arange(F, dtype=jnp.int32)[None, :]
    addr = (
        ((fi * (MAX_SIZE // 8) + x // 8) * (B // 128) + bi // 128) * 1024
        + (x % 8) * 128
        + (bi % 128)
    )
    idx = (
        addr.reshape(ns, b_per_s, nc, f_per_c)
        .transpose(2, 0, 1, 3)
        .reshape(nw, irows, 128)
    )

    mesh = plsc.VectorSubcoreMesh(
        core_axis_name="c", subcore_axis_name="s", num_cores=nc, num_subcores=ns
    )

    @functools.partial(
        pl.kernel,
        mesh=mesh,
        compiler_params=pltpu.CompilerParams(
            needs_layout_passes=False, use_tc_tiling_on_sc=False
        ),
        out_type=jax.ShapeDtypeStruct((total,), jnp.float32),
        scratch_types=[
            pltpu.VMEM((ZCHUNK,), jnp.float32),
            pltpu.VMEM((128,), jnp.float32),
            pltpu.VMEM((irows, 128), jnp.int32),
            pltpu.SemaphoreType.DMA,
            pltpu.SemaphoreType.DMA,
        ],
    )
    def onehot(zeros_hbm, idx_hbm, out_hbm, zbuf, ones_v, idx_v, zsem, ssem):
        c = lax.axis_index("c")
        s = lax.axis_index("s")
        wid = c * ns + s
        base = wid * words_pw

        pltpu.sync_copy(zeros_hbm, zbuf)
        pltpu.sync_copy(idx_hbm.at[wid], idx_v)
        ones16 = jnp.ones((16,), jnp.float32)
        for j in range(128 // 16):
            ones_v[pl.ds(j * 16, 16)] = ones16

        def zwait():
            pltpu.make_async_copy(zbuf, out_hbm.at[pl.ds(0, ZCHUNK)], zsem).wait()

        for g in range(nz // DRAIN):
            for k in range(DRAIN):
                pltpu.async_copy(
                    zbuf,
                    out_hbm.at[pl.ds(base + (g * DRAIN + k) * ZCHUNK, ZCHUNK)],
                    zsem,
                )
            for k in range(DRAIN):
                zwait()

        plsc.subcore_barrier()

        for j in range(irows):
            pltpu.async_copy(ones_v, out_hbm.at[idx_v.at[j]], ssem)
        for j in range(irows):
            pltpu.make_async_copy(ones_v, out_hbm.at[idx_v.at[0]], ssem).wait()

    zeros_buf = jnp.zeros((ZCHUNK,), jnp.float32)
    p = onehot(zeros_buf, idx)
    return (
        p.reshape(F, MAX_SIZE // 8, B // 128, 8, 128)
        .transpose(2, 4, 0, 1, 3)
        .reshape(B, F, MAX_SIZE)
    )


# speedup vs baseline: 1.0559x; 1.0559x over previous
import functools
import jax
import jax.numpy as jnp
from jax import lax
from jax.experimental import pallas as pl
from jax.experimental.pallas import tpu as pltpu
from jax.experimental.pallas import tpu_sc as plsc

MAX_SIZE = 1000
ZBLOCK = 1024000


def _zero_block(o_ref):
    o_ref[...] = jnp.zeros((ZBLOCK,), jnp.float32)


def kernel(x):
    B, F = x.shape
    nc, ns = 2, 16
    nw = nc * ns
    n = B * F
    total = n * MAX_SIZE
    pairs_pw = n // nw
    irows = pairs_pw // 128

    bi = jnp.arange(B, dtype=jnp.int32)[:, None]
    fi = jnp.arange(F, dtype=jnp.int32)[None, :]
    addr = (
        ((fi * (MAX_SIZE // 8) + x // 8) * (B // 128) + bi // 128) * 1024
        + (x % 8) * 128
        + (bi % 128)
    )
    idx = addr.reshape(nw, irows, 128)

    zeros = pl.pallas_call(
        _zero_block,
        grid=(total // ZBLOCK,),
        out_specs=pl.BlockSpec((ZBLOCK,), lambda i: (i,)),
        out_shape=jax.ShapeDtypeStruct((total,), jnp.float32),
    )()

    mesh = plsc.VectorSubcoreMesh(
        core_axis_name="c", subcore_axis_name="s", num_cores=nc, num_subcores=ns
    )

    @functools.partial(
        pl.kernel,
        mesh=mesh,
        compiler_params=pltpu.CompilerParams(
            needs_layout_passes=False, use_tc_tiling_on_sc=False
        ),
        scratch_types=[
            pltpu.VMEM((128,), jnp.float32),
            pltpu.VMEM((irows, 128), jnp.int32),
            pltpu.SemaphoreType.DMA,
        ],
    )
    def scatter_ones(idx_hbm, out_ref, ones_v, idx_v, ssem):
        c = lax.axis_index("c")
        s = lax.axis_index("s")
        wid = c * ns + s
        pltpu.sync_copy(idx_hbm.at[wid], idx_v)
        ones16 = jnp.ones((16,), jnp.float32)
        for j in range(128 // 16):
            ones_v[pl.ds(j * 16, 16)] = ones16
        for j in range(irows):
            pltpu.async_copy(ones_v, out_ref.at[idx_v.at[j]], ssem)
        for j in range(irows):
            pltpu.make_async_copy(ones_v, out_ref.at[idx_v.at[0]], ssem).wait()

    out_ref = jax.new_ref(zeros)
    scatter_ones(idx, out_ref)
    p = out_ref[...]
    return (
        p.reshape(F, MAX_SIZE // 8, B // 128, 8, 128)
        .transpose(2, 4, 0, 1, 3)
        .reshape(B, F, MAX_SIZE)
    )


# speedup vs baseline: 1.5230x; 1.4424x over previous
import functools
import jax
import jax.numpy as jnp
from jax import lax
from jax.experimental import pallas as pl
from jax.experimental.pallas import tpu as pltpu
from jax.experimental.pallas import tpu_sc as plsc

MAX_SIZE = 1000
FSPLIT = 20


def _tc_block(x_ref, o_ref):
    F = o_ref.shape[0]
    xv = x_ref[0]
    shape = (F, MAX_SIZE // 8, 1, 8, 128)
    v = lax.broadcasted_iota(jnp.int32, shape, 1) * 8 + lax.broadcasted_iota(
        jnp.int32, shape, 3
    )
    fi = lax.broadcasted_iota(jnp.int32, shape, 0)
    hit = (xv[:, None, None, None, :] == v) & (fi < FSPLIT)
    o_ref[...] = hit.astype(jnp.float32)


def kernel(x):
    B, F = x.shape
    nc, ns = 2, 16
    nw = nc * ns
    total = B * F * MAX_SIZE
    nbt = B // 128
    f_sc = F - FSPLIT

    xs = x[:, FSPLIT:]
    bi = jnp.arange(B, dtype=jnp.int32)[:, None]
    fi = jnp.arange(FSPLIT, F, dtype=jnp.int32)[None, :]
    addr = (
        ((fi * (MAX_SIZE // 8) + xs // 8) * (B // 128) + bi // 128) * 1024
        + (xs % 8) * 128
        + (bi % 128)
    )
    idx = addr.reshape(nw, 128, f_sc).transpose(0, 2, 1)

    x3 = x.reshape(nbt, 128, F).transpose(0, 2, 1)
    dense = pl.pallas_call(
        _tc_block,
        grid=(nbt,),
        in_specs=[pl.BlockSpec((1, F, 128), lambda i: (i, 0, 0))],
        out_specs=pl.BlockSpec(
            (F, MAX_SIZE // 8, 1, 8, 128), lambda i: (0, 0, i, 0, 0)
        ),
        out_shape=jax.ShapeDtypeStruct(
            (F, MAX_SIZE // 8, nbt, 8, 128), jnp.float32
        ),
    )(x3)

    mesh = plsc.VectorSubcoreMesh(
        core_axis_name="c", subcore_axis_name="s", num_cores=nc, num_subcores=ns
    )

    @functools.partial(
        pl.kernel,
        mesh=mesh,
        compiler_params=pltpu.CompilerParams(
            needs_layout_passes=False, use_tc_tiling_on_sc=False
        ),
        scratch_types=[
            pltpu.VMEM((128,), jnp.float32),
            pltpu.VMEM((f_sc, 128), jnp.int32),
            pltpu.SemaphoreType.DMA,
        ],
    )
    def scatter_ones(idx_hbm, out_ref, ones_v, idx_v, ssem):
        c = lax.axis_index("c")
        s = lax.axis_index("s")
        wid = c * ns + s
        pltpu.sync_copy(idx_hbm.at[wid], idx_v)
        ones16 = jnp.ones((16,), jnp.float32)
        for j in range(128 // 16):
            ones_v[pl.ds(j * 16, 16)] = ones16
        for j in range(f_sc):
            pltpu.async_copy(ones_v, out_ref.at[idx_v.at[j]], ssem)
        for j in range(f_sc):
            pltpu.make_async_copy(ones_v, out_ref.at[idx_v.at[0]], ssem).wait()

    out_ref = jax.new_ref(dense.reshape(total))
    scatter_ones(idx, out_ref)
    p = out_ref[...]
    return (
        p.reshape(F, MAX_SIZE // 8, B // 128, 8, 128)
        .transpose(2, 4, 0, 1, 3)
        .reshape(B, F, MAX_SIZE)
    )
